# baseline (speedup 1.0000x reference)
import math

import jax
import jax.numpy as jnp
from jax import lax
from jax.experimental import pallas as pl
from jax.experimental.pallas import tpu as pltpu
from jax.experimental.pallas import tpu_sc as plsc

D_MODEL = 128
_SCALE = math.sqrt(float(D_MODEL))

_NC = 2
_NS = 16
_NW = _NC * _NS

_NROWS = 4096
_K = 50
_CPW = _NROWS // _NW
_NBUF = 5

_mesh = plsc.VectorSubcoreMesh(core_axis_name="c", subcore_axis_name="s")


@pl.kernel(
    out_type=jax.ShapeDtypeStruct((_K, _NROWS, D_MODEL), jnp.float32),
    mesh=_mesh,
    scratch_types=(
        [pltpu.VMEM((_K, _CPW), jnp.int32)]
        + [pltpu.VMEM((_CPW, D_MODEL), jnp.float32) for _ in range(_NBUF)]
        + [pltpu.SemaphoreType.DMA for _ in range(2 * _NBUF)]
    ),
    compiler_params=pltpu.CompilerParams(use_tc_tiling_on_sc=True),
)
def _gather_scale(idx_hbm, table_hbm, out_hbm, idx_v, r0, r1, r2, r3, r4,
                  g0, g1, g2, g3, g4, s0, s1, s2, s3, s4):
    bufs = (r0, r1, r2, r3, r4)
    gsems = (g0, g1, g2, g3, g4)
    ssems = (s0, s1, s2, s3, s4)
    wid = lax.axis_index("s") * _NC + lax.axis_index("c")
    base_col = wid * _CPW
    scale = jnp.float32(_SCALE)

    pltpu.sync_copy(
        idx_hbm.at[pl.ds(0, _K), pl.ds(base_col, _CPW)], idx_v)

    def start_gather(k, b):
        pltpu.async_copy(table_hbm.at[idx_v.at[k]], bufs[b], gsems[b])

    def wait_gather(b):
        pltpu.make_async_copy(
            table_hbm.at[idx_v.at[0]], bufs[b], gsems[b]).wait()

    def wait_store(b):
        pltpu.make_async_copy(
            bufs[b], out_hbm.at[0, pl.ds(base_col, _CPW)], ssems[b]).wait()

    for c in range(_NBUF - 1):
        start_gather(c, c)

    @pl.loop(0, _K, step=_NBUF)
    def _chunks(j):
        for b in range(_NBUF):
            k = j + b
            nb = (b + _NBUF - 1) % _NBUF
            @pl.when(k + _NBUF - 1 < _K)
            def _():
                @pl.when(k >= 1)
                def _():
                    wait_store(nb)
                start_gather(k + _NBUF - 1, nb)

            wait_gather(b)

            @plsc.parallel_loop(0, _CPW, unroll=2)
            def _rows(i):
                for q in range(D_MODEL // 16):
                    sl = pl.ds(q * 16, 16)
                    bufs[b][i, sl] = bufs[b][i, sl] * scale

            pltpu.async_copy(
                bufs[b],
                out_hbm.at[k, pl.ds(base_col, _CPW)],
                ssems[b])

    for c in range(_K - _NBUF, _K):
        wait_store(c % _NBUF)


def kernel(idxs, emb_table):
    out_t = _gather_scale(idxs.T.astype(jnp.int32), emb_table)
    return out_t.transpose(1, 0, 2)


# speedup vs baseline: 1.0103x; 1.0103x over previous
import math

import jax
import jax.numpy as jnp
from jax import lax
from jax.experimental import pallas as pl
from jax.experimental.pallas import tpu as pltpu
from jax.experimental.pallas import tpu_sc as plsc

D_MODEL = 128
_SCALE = math.sqrt(float(D_MODEL))

_NC = 2
_NS = 16
_NW = _NC * _NS

_NROWS = 4096
_K = 50
_CPW = _NROWS // _NW
_CH = 64
_NCHUNK = _K * _CPW // _CH
_NBUF = 10

_mesh = plsc.VectorSubcoreMesh(core_axis_name="c", subcore_axis_name="s")


@pl.kernel(
    out_type=jax.ShapeDtypeStruct((_K, _NROWS, D_MODEL), jnp.float32),
    mesh=_mesh,
    scratch_types=(
        [pltpu.VMEM((_K, _CPW), jnp.int32)]
        + [pltpu.VMEM((_CH, D_MODEL), jnp.float32) for _ in range(_NBUF)]
        + [pltpu.SemaphoreType.DMA for _ in range(2 * _NBUF)]
    ),
    compiler_params=pltpu.CompilerParams(use_tc_tiling_on_sc=True),
)
def _gather_scale(idx_hbm, table_hbm, out_hbm, idx_v, *bs):
    bufs = bs[:_NBUF]
    gsems = bs[_NBUF:2 * _NBUF]
    ssems = bs[2 * _NBUF:]
    wid = lax.axis_index("s") * _NC + lax.axis_index("c")
    base_col = wid * _CPW
    scale = jnp.float32(_SCALE)

    pltpu.sync_copy(
        idx_hbm.at[pl.ds(0, _K), pl.ds(base_col, _CPW)], idx_v)

    def start_gather(c, b):
        kk = c >> 1
        half = (c & 1) * _CH
        pltpu.async_copy(
            table_hbm.at[idx_v.at[kk, pl.ds(half, _CH)]], bufs[b], gsems[b])

    def wait_gather(b):
        pltpu.make_async_copy(
            table_hbm.at[idx_v.at[0, pl.ds(0, _CH)]], bufs[b], gsems[b]).wait()

    def wait_store(b):
        pltpu.make_async_copy(
            bufs[b], out_hbm.at[0, pl.ds(base_col, _CH)], ssems[b]).wait()

    for c in range(_NBUF - 1):
        start_gather(c, c)

    @pl.loop(0, _NCHUNK, step=_NBUF)
    def _chunks(j):
        for b in range(_NBUF):
            c = j + b
            nb = (b + _NBUF - 1) % _NBUF
            @pl.when(c + _NBUF - 1 < _NCHUNK)
            def _():
                @pl.when(c >= 1)
                def _():
                    wait_store(nb)
                start_gather(c + _NBUF - 1, nb)

            wait_gather(b)

            @plsc.parallel_loop(0, _CH, unroll=2)
            def _rows(i):
                for q in range(D_MODEL // 16):
                    sl = pl.ds(q * 16, 16)
                    bufs[b][i, sl] = bufs[b][i, sl] * scale

            kk = c >> 1
            half = (c & 1) * _CH
            pltpu.async_copy(
                bufs[b],
                out_hbm.at[kk, pl.ds(base_col + half, _CH)],
                ssems[b])

    for c in range(_NCHUNK - _NBUF, _NCHUNK):
        wait_store(c % _NBUF)


def kernel(idxs, emb_table):
    out_t = _gather_scale(idxs.T.astype(jnp.int32), emb_table)
    return out_t.transpose(1, 0, 2)
